# baseline (speedup 1.0000x reference)
import jax
import jax.numpy as jnp
from jax import lax
from jax.experimental import pallas as pl
from jax.experimental.pallas import tpu as pltpu
from jax.experimental.pallas import tpu_sc as plsc

N, E, D, H, R = 10000, 320000, 128, 128, 16

NC, NS, LANES = 2, 16, 16
NW = NC * NS
CHUNK = 128
CPW0 = 160
CPW1 = 0
E_PAD = NS * (CPW0 + CPW1) * CHUNK
NPAD = 10112
RPT = NPAD // NS


_BT = 2000


def _transform_body(nf_ref, rw_ref, out_ref):
    out_ref[0] = jnp.dot(nf_ref[...], rw_ref[0], preferred_element_type=jnp.float32)


def _transform(node_feats, rel_weights, interpret=False):
    nb = pl.cdiv(N, _BT)
    return pl.pallas_call(
        _transform_body,
        grid=(R, nb),
        in_specs=[
            pl.BlockSpec((_BT, D), lambda r, i: (i, 0)),
            pl.BlockSpec((1, D, H), lambda r, i: (r, 0, 0)),
        ],
        out_specs=pl.BlockSpec((1, _BT, H), lambda r, i: (r, i, 0)),
        out_shape=jax.ShapeDtypeStruct((R, N, H), jnp.float32),
        interpret=interpret,
    )(node_feats, rel_weights)


def _sc_agg_body(table, src, ety, dst, sum_out, cnt_out,
                 src_v0, ety_v0, dst_v0, gidx_v0,
                 src_v1, ety_v1, dst_v1, gidx_v1,
                 rows_v0, rows_v1, cnt_hist, ssum, semg0, semg1):
    cid = lax.axis_index("c")
    sid = lax.axis_index("s")
    w = cid * NS + sid
    bufs = ((src_v0, ety_v0, dst_v0, gidx_v0, rows_v0, semg0),
            (src_v1, ety_v1, dst_v1, gidx_v1, rows_v1, semg1))

    zero16f = jnp.zeros((LANES,), jnp.float32)
    one16f = jnp.ones((LANES,), jnp.float32)

    def zero_row(i, carry):
        for t in range(H // LANES):
            rows_v0[i, pl.ds(t * LANES, LANES)] = zero16f
        return carry

    lax.fori_loop(0, CHUNK, zero_row, 0)

    def zero_hist(i, carry):
        cnt_hist[pl.ds(i * LANES, LANES)] = zero16f
        return carry

    lax.fori_loop(0, NPAD // LANES, zero_hist, 0)

    for k in range(RPT // CHUNK):
        base = sid * RPT + k * CHUNK
        pltpu.sync_copy(rows_v0, ssum.at[pl.ds(base, CHUNK)])
    rem = RPT % CHUNK
    if rem:
        base = sid * RPT + (RPT // CHUNK) * CHUNK
        pltpu.sync_copy(rows_v0.at[pl.ds(0, rem)], ssum.at[pl.ds(base, rem)])
    plsc.subcore_barrier()

    cpw = jnp.where(cid == 0, CPW0, CPW1)
    ebase = jnp.where(cid == 0, sid * (CPW0 * CHUNK),
                      NS * (CPW0 * CHUNK) + sid * (CPW1 * CHUNK))

    def load_and_start(j, b):
        sv, ev, dv, gv, rv, sg = bufs[b]
        off = ebase + j * CHUNK
        pltpu.sync_copy(src.at[pl.ds(off, CHUNK)], sv)
        pltpu.sync_copy(ety.at[pl.ds(off, CHUNK)], ev)
        pltpu.sync_copy(dst.at[pl.ds(off, CHUNK)], dv)
        for t in range(CHUNK // LANES):
            s16 = sv[pl.ds(t * LANES, LANES)]
            e16 = ev[pl.ds(t * LANES, LANES)]
            gv[pl.ds(t * LANES, LANES)] = e16 * N + s16
        pltpu.async_copy(table.at[gv], rv, sg)

    @pl.when(cpw >= 1)
    def _():
        load_and_start(0, 0)

    def outer(jo, carry):
        for b in range(2):
            j = 2 * jo + b
            sv, ev, dv, gv, rv, sg = bufs[b]

            @pl.when(j + 1 <= cpw - 1)
            def _():
                load_and_start(j + 1, 1 - b)

            pltpu.make_async_copy(table.at[gv], rv, sg).wait()
            pltpu.sync_copy(rv, ssum.at[dv], add=True)
            for t in range(CHUNK // LANES):
                plsc.addupdate_scatter(cnt_hist, [dv[pl.ds(t * LANES, LANES)]], one16f)
        return carry

    lax.fori_loop(0, cpw // 2, outer, 0)
    plsc.subcore_barrier()

    rbase = sid * RPT
    obase = cid * NPAD + sid * RPT
    pltpu.sync_copy(ssum.at[pl.ds(rbase, RPT)], sum_out.at[pl.ds(obase, RPT)])
    pltpu.sync_copy(cnt_hist, cnt_out.at[pl.ds(w * NPAD, NPAD)])


import functools


@functools.lru_cache(maxsize=None)
def _make_sc_agg():
    return pl.kernel(
        _sc_agg_body,
        out_type=(
            jax.ShapeDtypeStruct((NC * NPAD, H), jnp.float32),
            jax.ShapeDtypeStruct((NW * NPAD,), jnp.float32),
        ),
        mesh=plsc.VectorSubcoreMesh(
            core_axis_name="c", subcore_axis_name="s", num_cores=NC, num_subcores=NS
        ),
        scratch_types=[
            pltpu.VMEM((CHUNK,), jnp.int32),
            pltpu.VMEM((CHUNK,), jnp.int32),
            pltpu.VMEM((CHUNK,), jnp.int32),
            pltpu.VMEM((CHUNK,), jnp.int32),
            pltpu.VMEM((CHUNK,), jnp.int32),
            pltpu.VMEM((CHUNK,), jnp.int32),
            pltpu.VMEM((CHUNK,), jnp.int32),
            pltpu.VMEM((CHUNK,), jnp.int32),
            pltpu.VMEM((CHUNK, H), jnp.float32),
            pltpu.VMEM((CHUNK, H), jnp.float32),
            pltpu.VMEM((NPAD,), jnp.float32),
            pltpu.VMEM_SHARED((NPAD, H), jnp.float32),
            pltpu.SemaphoreType.DMA,
            pltpu.SemaphoreType.DMA,
        ],
        compiler_params=pltpu.CompilerParams(needs_layout_passes=False),
    )


_BF = NPAD


def _final_body(s0_ref, s1_ref, c_ref, nf_ref, w_ref, b_ref, out_ref):
    ones_nw = jnp.ones((NW, 1), jnp.float32)
    cnt = lax.dot_general(c_ref[...], ones_nw, (((0,), (0,)), ((), ())),
                          preferred_element_type=jnp.float32)
    cnt = jnp.maximum(cnt, 1.0)
    mean = (s0_ref[...] + s1_ref[...]) / cnt
    lin = jnp.dot(nf_ref[...], w_ref[...], preferred_element_type=jnp.float32) + b_ref[...]
    out_ref[...] = jnp.tanh(mean + lin)


def _final(sums, cnts, node_feats, lin_w, lin_b2, interpret=False):
    nb = pl.cdiv(N, _BF)
    off = NPAD // _BF
    return pl.pallas_call(
        _final_body,
        grid=(nb,),
        in_specs=[
            pl.BlockSpec((_BF, H), lambda i: (i, 0)),
            pl.BlockSpec((_BF, H), lambda i, o=off: (i + o, 0)),
            pl.BlockSpec((NW, _BF), lambda i: (0, i)),
            pl.BlockSpec((_BF, D), lambda i: (i, 0)),
            pl.BlockSpec((D, H), lambda i: (0, 0)),
            pl.BlockSpec((1, H), lambda i: (0, 0)),
        ],
        out_specs=pl.BlockSpec((_BF, H), lambda i: (i, 0)),
        out_shape=jax.ShapeDtypeStruct((N, H), jnp.float32),
        interpret=interpret,
    )(sums, sums, cnts, node_feats, lin_w, lin_b2)


def kernel(node_feats, edge_index, edge_types, rel_weights, lin_w, lin_b):
    src = edge_index[0]
    dst = edge_index[1]
    pad = E_PAD - E
    src_p = jnp.concatenate([src, jnp.zeros((pad,), jnp.int32)])
    ety_p = jnp.concatenate([edge_types, jnp.zeros((pad,), jnp.int32)])
    dummy_dst = N + jnp.arange(pad, dtype=jnp.int32) % (NPAD - N)
    dst_p = jnp.concatenate([dst, dummy_dst])

    table = _transform(node_feats, rel_weights).reshape(R * N, H)
    sums, cnts = _make_sc_agg()(table, src_p, ety_p, dst_p)
    return _final(sums, cnts.reshape(NW, NPAD), node_feats, lin_w,
                  lin_b.reshape(1, H))


# speedup vs baseline: 1.3344x; 1.3344x over previous
import jax
import jax.numpy as jnp
from jax import lax
from jax.experimental import pallas as pl
from jax.experimental.pallas import tpu as pltpu
from jax.experimental.pallas import tpu_sc as plsc

N, E, D, H, R = 10000, 320000, 128, 128, 16

NC, NS, LANES = 2, 16, 16
NW = NC * NS
CHUNK = 128
CPW0 = 96
CPW1 = 64
E_PAD = NS * (CPW0 + CPW1) * CHUNK
NPAD = 10112
RPT = NPAD // NS


_BT = 2000


def _transform_body(nf_ref, rw_ref, out_ref):
    out_ref[0] = jnp.dot(nf_ref[...], rw_ref[0], preferred_element_type=jnp.float32)


def _transform(node_feats, rel_weights, interpret=False):
    nb = pl.cdiv(N, _BT)
    return pl.pallas_call(
        _transform_body,
        grid=(R, nb),
        in_specs=[
            pl.BlockSpec((_BT, D), lambda r, i: (i, 0)),
            pl.BlockSpec((1, D, H), lambda r, i: (r, 0, 0)),
        ],
        out_specs=pl.BlockSpec((1, _BT, H), lambda r, i: (r, i, 0)),
        out_shape=jax.ShapeDtypeStruct((R, N, H), jnp.float32),
        interpret=interpret,
    )(node_feats, rel_weights)


def _sc_agg_body(table, src, ety, dst, sum_out, cnt_out,
                 src_v0, ety_v0, dst_v0, gidx_v0,
                 src_v1, ety_v1, dst_v1, gidx_v1,
                 rows_v0, rows_v1, cnt_hist, ssum, semg0, semg1):
    cid = lax.axis_index("c")
    sid = lax.axis_index("s")
    w = cid * NS + sid
    bufs = ((src_v0, ety_v0, dst_v0, gidx_v0, rows_v0, semg0),
            (src_v1, ety_v1, dst_v1, gidx_v1, rows_v1, semg1))

    zero16f = jnp.zeros((LANES,), jnp.float32)
    one16f = jnp.ones((LANES,), jnp.float32)

    def zero_row(i, carry):
        for t in range(H // LANES):
            rows_v0[i, pl.ds(t * LANES, LANES)] = zero16f
        return carry

    lax.fori_loop(0, CHUNK, zero_row, 0)

    def zero_hist(i, carry):
        cnt_hist[pl.ds(i * LANES, LANES)] = zero16f
        return carry

    lax.fori_loop(0, NPAD // LANES, zero_hist, 0)

    for k in range(RPT // CHUNK):
        base = sid * RPT + k * CHUNK
        pltpu.sync_copy(rows_v0, ssum.at[pl.ds(base, CHUNK)])
    rem = RPT % CHUNK
    if rem:
        base = sid * RPT + (RPT // CHUNK) * CHUNK
        pltpu.sync_copy(rows_v0.at[pl.ds(0, rem)], ssum.at[pl.ds(base, rem)])
    plsc.subcore_barrier()

    cpw = jnp.where(cid == 0, CPW0, CPW1)
    ebase = jnp.where(cid == 0, sid * (CPW0 * CHUNK),
                      NS * (CPW0 * CHUNK) + sid * (CPW1 * CHUNK))

    def load_and_start(j, b):
        sv, ev, dv, gv, rv, sg = bufs[b]
        off = ebase + j * CHUNK
        pltpu.sync_copy(src.at[pl.ds(off, CHUNK)], sv)
        pltpu.sync_copy(ety.at[pl.ds(off, CHUNK)], ev)
        pltpu.sync_copy(dst.at[pl.ds(off, CHUNK)], dv)
        for t in range(CHUNK // LANES):
            s16 = sv[pl.ds(t * LANES, LANES)]
            e16 = ev[pl.ds(t * LANES, LANES)]
            gv[pl.ds(t * LANES, LANES)] = e16 * N + s16
        pltpu.async_copy(table.at[gv], rv, sg)

    @pl.when(cpw >= 1)
    def _():
        load_and_start(0, 0)

    def outer(jo, carry):
        for b in range(2):
            j = 2 * jo + b
            sv, ev, dv, gv, rv, sg = bufs[b]

            @pl.when(j + 1 <= cpw - 1)
            def _():
                load_and_start(j + 1, 1 - b)

            pltpu.make_async_copy(table.at[gv], rv, sg).wait()
            pltpu.sync_copy(rv, ssum.at[dv], add=True)
            for t in range(CHUNK // LANES):
                plsc.addupdate_scatter(cnt_hist, [dv[pl.ds(t * LANES, LANES)]], one16f)
        return carry

    lax.fori_loop(0, cpw // 2, outer, 0)
    plsc.subcore_barrier()

    rbase = sid * RPT
    obase = cid * NPAD + sid * RPT
    pltpu.sync_copy(ssum.at[pl.ds(rbase, RPT)], sum_out.at[pl.ds(obase, RPT)])
    pltpu.sync_copy(cnt_hist, cnt_out.at[pl.ds(w * NPAD, NPAD)])


import functools


@functools.lru_cache(maxsize=None)
def _make_sc_agg():
    return pl.kernel(
        _sc_agg_body,
        out_type=(
            jax.ShapeDtypeStruct((NC * NPAD, H), jnp.float32),
            jax.ShapeDtypeStruct((NW * NPAD,), jnp.float32),
        ),
        mesh=plsc.VectorSubcoreMesh(
            core_axis_name="c", subcore_axis_name="s", num_cores=NC, num_subcores=NS
        ),
        scratch_types=[
            pltpu.VMEM((CHUNK,), jnp.int32),
            pltpu.VMEM((CHUNK,), jnp.int32),
            pltpu.VMEM((CHUNK,), jnp.int32),
            pltpu.VMEM((CHUNK,), jnp.int32),
            pltpu.VMEM((CHUNK,), jnp.int32),
            pltpu.VMEM((CHUNK,), jnp.int32),
            pltpu.VMEM((CHUNK,), jnp.int32),
            pltpu.VMEM((CHUNK,), jnp.int32),
            pltpu.VMEM((CHUNK, H), jnp.float32),
            pltpu.VMEM((CHUNK, H), jnp.float32),
            pltpu.VMEM((NPAD,), jnp.float32),
            pltpu.VMEM_SHARED((NPAD, H), jnp.float32),
            pltpu.SemaphoreType.DMA,
            pltpu.SemaphoreType.DMA,
        ],
        compiler_params=pltpu.CompilerParams(needs_layout_passes=False),
    )


_BF = NPAD


def _final_body(s0_ref, s1_ref, c_ref, nf_ref, w_ref, b_ref, out_ref):
    ones_nw = jnp.ones((NW, 1), jnp.float32)
    cnt = lax.dot_general(c_ref[...], ones_nw, (((0,), (0,)), ((), ())),
                          preferred_element_type=jnp.float32)
    cnt = jnp.maximum(cnt, 1.0)
    mean = (s0_ref[...] + s1_ref[...]) / cnt
    lin = jnp.dot(nf_ref[...], w_ref[...], preferred_element_type=jnp.float32) + b_ref[...]
    out_ref[...] = jnp.tanh(mean + lin)


def _final(sums, cnts, node_feats, lin_w, lin_b2, interpret=False):
    nb = pl.cdiv(N, _BF)
    off = NPAD // _BF
    return pl.pallas_call(
        _final_body,
        grid=(nb,),
        in_specs=[
            pl.BlockSpec((_BF, H), lambda i: (i, 0)),
            pl.BlockSpec((_BF, H), lambda i, o=off: (i + o, 0)),
            pl.BlockSpec((NW, _BF), lambda i: (0, i)),
            pl.BlockSpec((_BF, D), lambda i: (i, 0)),
            pl.BlockSpec((D, H), lambda i: (0, 0)),
            pl.BlockSpec((1, H), lambda i: (0, 0)),
        ],
        out_specs=pl.BlockSpec((_BF, H), lambda i: (i, 0)),
        out_shape=jax.ShapeDtypeStruct((N, H), jnp.float32),
        interpret=interpret,
    )(sums, sums, cnts, node_feats, lin_w, lin_b2)


def kernel(node_feats, edge_index, edge_types, rel_weights, lin_w, lin_b):
    src = edge_index[0]
    dst = edge_index[1]
    pad = E_PAD - E
    src_p = jnp.concatenate([src, jnp.zeros((pad,), jnp.int32)])
    ety_p = jnp.concatenate([edge_types, jnp.zeros((pad,), jnp.int32)])
    dummy_dst = N + jnp.arange(pad, dtype=jnp.int32) % (NPAD - N)
    dst_p = jnp.concatenate([dst, dummy_dst])

    table = _transform(node_feats, rel_weights).reshape(R * N, H)
    sums, cnts = _make_sc_agg()(table, src_p, ety_p, dst_p)
    return _final(sums, cnts.reshape(NW, NPAD), node_feats, lin_w,
                  lin_b.reshape(1, H))


# speedup vs baseline: 1.3429x; 1.0064x over previous
import jax
import jax.numpy as jnp
from jax import lax
from jax.experimental import pallas as pl
from jax.experimental.pallas import tpu as pltpu
from jax.experimental.pallas import tpu_sc as plsc

N, E, D, H, R = 10000, 320000, 128, 128, 16

NC, NS, LANES = 2, 16, 16
NW = NC * NS
CHUNK = 128
CPW0 = 112
CPW1 = 48
E_PAD = NS * (CPW0 + CPW1) * CHUNK
NPAD = 10112
RPT = NPAD // NS


_BT = 2000


def _transform_body(nf_ref, rw_ref, out_ref):
    out_ref[0] = jnp.dot(nf_ref[...], rw_ref[0], preferred_element_type=jnp.float32)


def _transform(node_feats, rel_weights, interpret=False):
    nb = pl.cdiv(N, _BT)
    return pl.pallas_call(
        _transform_body,
        grid=(R, nb),
        in_specs=[
            pl.BlockSpec((_BT, D), lambda r, i: (i, 0)),
            pl.BlockSpec((1, D, H), lambda r, i: (r, 0, 0)),
        ],
        out_specs=pl.BlockSpec((1, _BT, H), lambda r, i: (r, i, 0)),
        out_shape=jax.ShapeDtypeStruct((R, N, H), jnp.float32),
        interpret=interpret,
    )(node_feats, rel_weights)


def _sc_agg_body(table, src, ety, dst, sum_out, cnt_out,
                 src_v0, ety_v0, dst_v0, gidx_v0,
                 src_v1, ety_v1, dst_v1, gidx_v1,
                 rows_v0, rows_v1, cnt_hist, ssum, semg0, semg1):
    cid = lax.axis_index("c")
    sid = lax.axis_index("s")
    w = cid * NS + sid
    bufs = ((src_v0, ety_v0, dst_v0, gidx_v0, rows_v0, semg0),
            (src_v1, ety_v1, dst_v1, gidx_v1, rows_v1, semg1))

    zero16f = jnp.zeros((LANES,), jnp.float32)
    one16f = jnp.ones((LANES,), jnp.float32)

    def zero_row(i, carry):
        for t in range(H // LANES):
            rows_v0[i, pl.ds(t * LANES, LANES)] = zero16f
        return carry

    lax.fori_loop(0, CHUNK, zero_row, 0)

    def zero_hist(i, carry):
        cnt_hist[pl.ds(i * LANES, LANES)] = zero16f
        return carry

    lax.fori_loop(0, NPAD // LANES, zero_hist, 0)

    for k in range(RPT // CHUNK):
        base = sid * RPT + k * CHUNK
        pltpu.sync_copy(rows_v0, ssum.at[pl.ds(base, CHUNK)])
    rem = RPT % CHUNK
    if rem:
        base = sid * RPT + (RPT // CHUNK) * CHUNK
        pltpu.sync_copy(rows_v0.at[pl.ds(0, rem)], ssum.at[pl.ds(base, rem)])
    plsc.subcore_barrier()

    cpw = jnp.where(cid == 0, CPW0, CPW1)
    ebase = jnp.where(cid == 0, sid * (CPW0 * CHUNK),
                      NS * (CPW0 * CHUNK) + sid * (CPW1 * CHUNK))

    def load_and_start(j, b):
        sv, ev, dv, gv, rv, sg = bufs[b]
        off = ebase + j * CHUNK
        pltpu.sync_copy(src.at[pl.ds(off, CHUNK)], sv)
        pltpu.sync_copy(ety.at[pl.ds(off, CHUNK)], ev)
        pltpu.sync_copy(dst.at[pl.ds(off, CHUNK)], dv)
        for t in range(CHUNK // LANES):
            s16 = sv[pl.ds(t * LANES, LANES)]
            e16 = ev[pl.ds(t * LANES, LANES)]
            gv[pl.ds(t * LANES, LANES)] = e16 * N + s16
        pltpu.async_copy(table.at[gv], rv, sg)

    @pl.when(cpw >= 1)
    def _():
        load_and_start(0, 0)

    def outer(jo, carry):
        for b in range(2):
            j = 2 * jo + b
            sv, ev, dv, gv, rv, sg = bufs[b]

            @pl.when(j + 1 <= cpw - 1)
            def _():
                load_and_start(j + 1, 1 - b)

            pltpu.make_async_copy(table.at[gv], rv, sg).wait()
            pltpu.sync_copy(rv, ssum.at[dv], add=True)
            for t in range(CHUNK // LANES):
                plsc.addupdate_scatter(cnt_hist, [dv[pl.ds(t * LANES, LANES)]], one16f)
        return carry

    lax.fori_loop(0, cpw // 2, outer, 0)
    plsc.subcore_barrier()

    rbase = sid * RPT
    obase = cid * NPAD + sid * RPT
    pltpu.sync_copy(ssum.at[pl.ds(rbase, RPT)], sum_out.at[pl.ds(obase, RPT)])
    pltpu.sync_copy(cnt_hist, cnt_out.at[pl.ds(w * NPAD, NPAD)])


import functools


@functools.lru_cache(maxsize=None)
def _make_sc_agg():
    return pl.kernel(
        _sc_agg_body,
        out_type=(
            jax.ShapeDtypeStruct((NC * NPAD, H), jnp.float32),
            jax.ShapeDtypeStruct((NW * NPAD,), jnp.float32),
        ),
        mesh=plsc.VectorSubcoreMesh(
            core_axis_name="c", subcore_axis_name="s", num_cores=NC, num_subcores=NS
        ),
        scratch_types=[
            pltpu.VMEM((CHUNK,), jnp.int32),
            pltpu.VMEM((CHUNK,), jnp.int32),
            pltpu.VMEM((CHUNK,), jnp.int32),
            pltpu.VMEM((CHUNK,), jnp.int32),
            pltpu.VMEM((CHUNK,), jnp.int32),
            pltpu.VMEM((CHUNK,), jnp.int32),
            pltpu.VMEM((CHUNK,), jnp.int32),
            pltpu.VMEM((CHUNK,), jnp.int32),
            pltpu.VMEM((CHUNK, H), jnp.float32),
            pltpu.VMEM((CHUNK, H), jnp.float32),
            pltpu.VMEM((NPAD,), jnp.float32),
            pltpu.VMEM_SHARED((NPAD, H), jnp.float32),
            pltpu.SemaphoreType.DMA,
            pltpu.SemaphoreType.DMA,
        ],
        compiler_params=pltpu.CompilerParams(needs_layout_passes=False),
    )


_BF = NPAD


def _final_body(s0_ref, s1_ref, c_ref, nf_ref, w_ref, b_ref, out_ref):
    ones_nw = jnp.ones((NW, 1), jnp.float32)
    cnt = lax.dot_general(c_ref[...], ones_nw, (((0,), (0,)), ((), ())),
                          preferred_element_type=jnp.float32)
    cnt = jnp.maximum(cnt, 1.0)
    mean = (s0_ref[...] + s1_ref[...]) / cnt
    lin = jnp.dot(nf_ref[...], w_ref[...], preferred_element_type=jnp.float32) + b_ref[...]
    out_ref[...] = jnp.tanh(mean + lin)


def _final(sums, cnts, node_feats, lin_w, lin_b2, interpret=False):
    nb = pl.cdiv(N, _BF)
    off = NPAD // _BF
    return pl.pallas_call(
        _final_body,
        grid=(nb,),
        in_specs=[
            pl.BlockSpec((_BF, H), lambda i: (i, 0)),
            pl.BlockSpec((_BF, H), lambda i, o=off: (i + o, 0)),
            pl.BlockSpec((NW, _BF), lambda i: (0, i)),
            pl.BlockSpec((_BF, D), lambda i: (i, 0)),
            pl.BlockSpec((D, H), lambda i: (0, 0)),
            pl.BlockSpec((1, H), lambda i: (0, 0)),
        ],
        out_specs=pl.BlockSpec((_BF, H), lambda i: (i, 0)),
        out_shape=jax.ShapeDtypeStruct((N, H), jnp.float32),
        interpret=interpret,
    )(sums, sums, cnts, node_feats, lin_w, lin_b2)


def kernel(node_feats, edge_index, edge_types, rel_weights, lin_w, lin_b):
    src = edge_index[0]
    dst = edge_index[1]
    pad = E_PAD - E
    src_p = jnp.concatenate([src, jnp.zeros((pad,), jnp.int32)])
    ety_p = jnp.concatenate([edge_types, jnp.zeros((pad,), jnp.int32)])
    dummy_dst = N + jnp.arange(pad, dtype=jnp.int32) % (NPAD - N)
    dst_p = jnp.concatenate([dst, dummy_dst])

    table = _transform(node_feats, rel_weights).reshape(R * N, H)
    sums, cnts = _make_sc_agg()(table, src_p, ety_p, dst_p)
    return _final(sums, cnts.reshape(NW, NPAD), node_feats, lin_w,
                  lin_b.reshape(1, H))
